# Initial kernel scaffold; baseline (speedup 1.0000x reference)
import jax
import jax.numpy as jnp
from jax.experimental import pallas as pl


def kernel(input, dialog_lengths, speakers, labels, rgcn_weight, rgcn_root, rgcn_bias, gcn_rel_w, gcn_rel_b, gcn_root_w, skip_w, skip_b, cls_w, cls_b):
    raise NotImplementedError("write your pallas kernel here")


# speedup vs baseline: 19.1448x; 19.1448x over previous
import functools

import numpy as np
import jax
import jax.numpy as jnp
from jax.experimental import pallas as pl
from jax.experimental.pallas import tpu as pltpu

W = 4
HALO = 2 * W


def _static_meta(B, N, T, nb):
    lengths = np.arange(B, dtype=np.int64)
    lens_node = np.repeat(lengths, lengths)[:N]
    starts = np.cumsum(lengths) - lengths
    starts_node = np.repeat(starts, lengths)[:N]
    pos = np.arange(N, dtype=np.int64) - starts_node

    g = (np.arange(nb)[:, None] * T + np.arange(T + 2 * HALO)[None, :]) - HALO
    ok = (g >= 0) & (g < N)
    gc = np.clip(g, 0, N - 1)
    posw = pos[gc]
    lenw = lens_node[gc]
    meta = np.zeros((nb, T + 2 * HALO, 16), dtype=np.float32)
    for d in range(-W, W + 1):
        meta[..., d + W] = (ok & (posw + d >= 0) & (posw + d < lenw)).astype(np.float32)
    meta[..., 11] = ok.astype(np.float32)
    return meta, gc, ok


def _diagcn_block(xm_ref, xc_ref, xp_ref, meta_ref,
                  rw_ref, root_ref, rb_ref, relw_ref, relb_ref, rootw_ref,
                  skw_ref, skb_ref, cw_ref, cb_ref,
                  out_ref, loss_ref, *, T, nb, inv_n):
    i = pl.program_id(0)
    e = jnp.concatenate(
        [xm_ref[T - HALO:, :], xc_ref[...], xp_ref[:HALO, :]], axis=0)
    meta = meta_ref[0]
    sp = meta[:, 9:10]
    xs = e * sp

    Th = T + 2 * W
    s_all = jnp.zeros((Th, e.shape[1]), jnp.float32)
    s_sp = jnp.zeros((Th, e.shape[1]), jnp.float32)
    c_sp = jnp.zeros((Th, 1), jnp.float32)
    c_all = jnp.zeros((Th, 1), jnp.float32)
    for d in range(-W, W + 1):
        m = meta[W:W + Th, d + W:d + W + 1]
        s_all = s_all + m * e[HALO + d - W:HALO + d - W + Th, :]
        s_sp = s_sp + m * xs[HALO + d - W:HALO + d - W + Th, :]
        c_sp = c_sp + m * sp[HALO + d - W:HALO + d - W + Th, :]
        c_all = c_all + m

    sp_h = sp[W:W + Th, :]
    sum1 = sp_h * s_sp
    cnt1 = sp_h * c_sp
    mean1 = sum1 / jnp.maximum(cnt1, 1.0)
    mean0 = (s_all - sum1) / jnp.maximum(c_all - cnt1, 1.0)
    xh = e[W:W + Th, :]

    dn_nn = (((1,), (0,)), ((), ()))
    dn_nt = (((1,), (1,)), ((), ()))
    h = (jax.lax.dot_general(mean0, rw_ref[0], dn_nn,
                             preferred_element_type=jnp.float32)
         + jax.lax.dot_general(mean1, rw_ref[1], dn_nn,
                               preferred_element_type=jnp.float32)
         + jax.lax.dot_general(xh, root_ref[...], dn_nn,
                               preferred_element_type=jnp.float32)
         + rb_ref[...])

    agg = jnp.zeros((T, e.shape[1]), jnp.float32)
    for d in range(-W, W + 1):
        m = meta[HALO:HALO + T, d + W:d + W + 1]
        agg = agg + m * h[W + d:W + d + T, :]
    hc = h[W:W + T, :]
    xc = e[HALO:HALO + T, :]

    h2 = (jax.lax.dot_general(agg, relw_ref[...], dn_nt,
                              preferred_element_type=jnp.float32)
          + relb_ref[...]
          + jax.lax.dot_general(hc, rootw_ref[...], dn_nt,
                                preferred_element_type=jnp.float32))
    z = (h2 + jax.lax.dot_general(xc, skw_ref[...], dn_nt,
                                  preferred_element_type=jnp.float32)
         + skb_ref[...])
    out = (jax.lax.dot_general(z, cw_ref[...], dn_nt,
                               preferred_element_type=jnp.float32)
           + cb_ref[...])
    out_ref[...] = out

    lane = jax.lax.broadcasted_iota(jnp.int32, out.shape, 1)
    neg = jnp.float32(-1e30)
    outm = jnp.where(lane < 6, out, neg)
    mx = jnp.max(outm, axis=1, keepdims=True)
    lse = mx + jnp.log(jnp.sum(jnp.exp(outm - mx), axis=1, keepdims=True))
    lab = meta[HALO:HALO + T, 10:11].astype(jnp.int32)
    rv = meta[HALO:HALO + T, 11:12]
    picked = jnp.sum(jnp.where(lane == lab, out, 0.0), axis=1, keepdims=True)
    partial = jnp.sum(rv * (lse - picked)) * inv_n

    @pl.when(i == 0)
    def _():
        loss_ref[0, 0] = partial

    @pl.when(i > 0)
    def _():
        loss_ref[0, 0] = loss_ref[0, 0] + partial


def kernel(input, dialog_lengths, speakers, labels, rgcn_weight, rgcn_root,
           rgcn_bias, gcn_rel_w, gcn_rel_b, gcn_root_w, skip_w, skip_b,
           cls_w, cls_b):
    B = dialog_lengths.shape[0]
    N, D = input.shape
    H = rgcn_root.shape[1]
    T = 2048
    nb = -(-N // T)
    npad = nb * T

    meta_np, gc, ok = _static_meta(B, N, T, nb)
    meta = jnp.asarray(meta_np)
    okf = jnp.asarray(ok.astype(np.float32))
    gci = jnp.asarray(gc.astype(np.int32))
    spw = speakers.astype(jnp.float32)[gci] * okf
    labw = labels.astype(jnp.float32)[gci] * okf
    meta = meta.at[..., 9].set(spw).at[..., 10].set(labw)

    xpad = jnp.zeros((npad, D), jnp.float32).at[:N].set(input)
    cls_w8 = jnp.zeros((8, H), jnp.float32).at[:6].set(cls_w)
    cls_b8 = jnp.zeros((1, 8), jnp.float32).at[0, :6].set(cls_b)

    row_spec = lambda f: pl.BlockSpec((T, D), lambda i: (f(i), 0))
    full = lambda a: pl.BlockSpec(a.shape, lambda i: (0,) * a.ndim)

    out, loss = pl.pallas_call(
        functools.partial(_diagcn_block, T=T, nb=nb, inv_n=1.0 / N),
        grid=(nb,),
        in_specs=[
            row_spec(lambda i: jnp.maximum(i - 1, 0)),
            row_spec(lambda i: i),
            row_spec(lambda i: jnp.minimum(i + 1, nb - 1)),
            pl.BlockSpec((1, T + 2 * HALO, 16), lambda i: (i, 0, 0)),
            full(rgcn_weight), full(rgcn_root),
            pl.BlockSpec((1, H), lambda i: (0, 0)),
            full(gcn_rel_w), pl.BlockSpec((1, H), lambda i: (0, 0)),
            full(gcn_root_w), full(skip_w),
            pl.BlockSpec((1, H), lambda i: (0, 0)),
            full(cls_w8), full(cls_b8),
        ],
        out_specs=[
            pl.BlockSpec((T, 8), lambda i: (i, 0)),
            pl.BlockSpec(memory_space=pltpu.SMEM),
        ],
        out_shape=[
            jax.ShapeDtypeStruct((npad, 8), jnp.float32),
            jax.ShapeDtypeStruct((1, 1), jnp.float32),
        ],
    )(xpad, xpad, xpad, meta, rgcn_weight, rgcn_root,
      rgcn_bias.reshape(1, H), gcn_rel_w, gcn_rel_b.reshape(1, H),
      gcn_root_w, skip_w, skip_b.reshape(1, H), cls_w8, cls_b8)

    return (out[:N, :6], loss[0, 0])


# speedup vs baseline: 33.6262x; 1.7564x over previous
import functools

import numpy as np
import jax
import jax.numpy as jnp
from jax.experimental import pallas as pl
from jax.experimental.pallas import tpu as pltpu

W = 4
HALO = 2 * W


def _static_meta(B, N, T, nb):
    lengths = np.arange(B, dtype=np.int64)
    lens_node = np.repeat(lengths, lengths)[:N]
    starts = np.cumsum(lengths) - lengths
    starts_node = np.repeat(starts, lengths)[:N]
    pos = np.arange(N, dtype=np.int64) - starts_node

    g = (np.arange(nb)[:, None] * T + np.arange(T + 2 * HALO)[None, :]) - HALO
    ok = (g >= 0) & (g < N)
    gc = np.clip(g, 0, N - 1)
    posw = pos[gc]
    lenw = lens_node[gc]
    meta = np.zeros((nb, T + 2 * HALO, 16), dtype=np.float32)
    for d in range(-W, W + 1):
        meta[..., d + W] = (ok & (posw + d >= 0) & (posw + d < lenw)).astype(np.float32)
    meta[..., 11] = ok.astype(np.float32)
    return meta, gc, ok


def _windows(v, T, nb, npad):
    vpad = jnp.zeros((npad + 2 * HALO,), v.dtype).at[HALO:HALO + v.shape[0]].set(v)
    main = vpad[HALO:HALO + nb * T].reshape(nb, T)
    left = vpad[:nb * T].reshape(nb, T)[:, :HALO]
    right = jnp.concatenate([vpad[HALO + T:], jnp.zeros((T,), v.dtype)])[
        :nb * T].reshape(nb, T)[:, :HALO]
    return jnp.concatenate([left, main, right], axis=1)


def _diagcn_block(xm_ref, xc_ref, xp_ref, meta_ref,
                  rw_ref, root_ref, rb_ref, relw_ref, relb_ref, rootw_ref,
                  skw_ref, skb_ref, cw_ref, cb_ref,
                  out_ref, loss_ref, *, T, nb, inv_n):
    i = pl.program_id(0)
    e = jnp.concatenate(
        [xm_ref[T - HALO:, :], xc_ref[...], xp_ref[:HALO, :]], axis=0)
    meta = meta_ref[0]
    sp = meta[:, 9:10]
    xs = e * sp

    Th = T + 2 * W
    s_all = jnp.zeros((Th, e.shape[1]), jnp.float32)
    s_sp = jnp.zeros((Th, e.shape[1]), jnp.float32)
    c_sp = jnp.zeros((Th, 1), jnp.float32)
    c_all = jnp.zeros((Th, 1), jnp.float32)
    for d in range(-W, W + 1):
        m = meta[W:W + Th, d + W:d + W + 1]
        s_all = s_all + m * e[HALO + d - W:HALO + d - W + Th, :]
        s_sp = s_sp + m * xs[HALO + d - W:HALO + d - W + Th, :]
        c_sp = c_sp + m * sp[HALO + d - W:HALO + d - W + Th, :]
        c_all = c_all + m

    sp_h = sp[W:W + Th, :]
    sum1 = sp_h * s_sp
    cnt1 = sp_h * c_sp
    mean1 = sum1 / jnp.maximum(cnt1, 1.0)
    mean0 = (s_all - sum1) / jnp.maximum(c_all - cnt1, 1.0)
    xh = e[W:W + Th, :]

    dn_nn = (((1,), (0,)), ((), ()))
    dn_nt = (((1,), (1,)), ((), ()))
    h = (jax.lax.dot_general(mean0, rw_ref[0], dn_nn,
                             preferred_element_type=jnp.float32)
         + jax.lax.dot_general(mean1, rw_ref[1], dn_nn,
                               preferred_element_type=jnp.float32)
         + jax.lax.dot_general(xh, root_ref[...], dn_nn,
                               preferred_element_type=jnp.float32)
         + rb_ref[...])

    agg = jnp.zeros((T, e.shape[1]), jnp.float32)
    for d in range(-W, W + 1):
        m = meta[HALO:HALO + T, d + W:d + W + 1]
        agg = agg + m * h[W + d:W + d + T, :]
    hc = h[W:W + T, :]
    xc = e[HALO:HALO + T, :]

    h2 = (jax.lax.dot_general(agg, relw_ref[...], dn_nt,
                              preferred_element_type=jnp.float32)
          + relb_ref[...]
          + jax.lax.dot_general(hc, rootw_ref[...], dn_nt,
                                preferred_element_type=jnp.float32))
    z = (h2 + jax.lax.dot_general(xc, skw_ref[...], dn_nt,
                                  preferred_element_type=jnp.float32)
         + skb_ref[...])
    out = (jax.lax.dot_general(z, cw_ref[...], dn_nt,
                               preferred_element_type=jnp.float32)
           + cb_ref[...])
    out_ref[...] = out

    lane = jax.lax.broadcasted_iota(jnp.int32, out.shape, 1)
    neg = jnp.float32(-1e30)
    outm = jnp.where(lane < 6, out, neg)
    mx = jnp.max(outm, axis=1, keepdims=True)
    lse = mx + jnp.log(jnp.sum(jnp.exp(outm - mx), axis=1, keepdims=True))
    lab = meta[HALO:HALO + T, 10:11].astype(jnp.int32)
    rv = meta[HALO:HALO + T, 11:12]
    picked = jnp.sum(jnp.where(lane == lab, out, 0.0), axis=1, keepdims=True)
    partial = jnp.sum(rv * (lse - picked)) * inv_n

    @pl.when(i == 0)
    def _():
        loss_ref[0, 0] = partial

    @pl.when(i > 0)
    def _():
        loss_ref[0, 0] = loss_ref[0, 0] + partial


def kernel(input, dialog_lengths, speakers, labels, rgcn_weight, rgcn_root,
           rgcn_bias, gcn_rel_w, gcn_rel_b, gcn_root_w, skip_w, skip_b,
           cls_w, cls_b):
    B = dialog_lengths.shape[0]
    N, D = input.shape
    H = rgcn_root.shape[1]
    T = 2048
    nb = -(-N // T)
    npad = nb * T

    meta_np, gc, ok = _static_meta(B, N, T, nb)
    okf = jnp.asarray(ok.astype(np.float32))
    spw = _windows(speakers.astype(jnp.float32), T, nb, npad) * okf
    labw = _windows(labels.astype(jnp.float32), T, nb, npad) * okf
    meta = jnp.concatenate(
        [jnp.asarray(meta_np[..., :9]), spw[..., None], labw[..., None],
         jnp.asarray(meta_np[..., 11:])], axis=2)

    xpad = jnp.zeros((npad, D), jnp.float32).at[:N].set(input)
    cls_w8 = jnp.zeros((8, H), jnp.float32).at[:6].set(cls_w)
    cls_b8 = jnp.zeros((1, 8), jnp.float32).at[0, :6].set(cls_b)

    row_spec = lambda f: pl.BlockSpec((T, D), lambda i: (f(i), 0))
    full = lambda a: pl.BlockSpec(a.shape, lambda i: (0,) * a.ndim)

    out, loss = pl.pallas_call(
        functools.partial(_diagcn_block, T=T, nb=nb, inv_n=1.0 / N),
        grid=(nb,),
        in_specs=[
            row_spec(lambda i: jnp.maximum(i - 1, 0)),
            row_spec(lambda i: i),
            row_spec(lambda i: jnp.minimum(i + 1, nb - 1)),
            pl.BlockSpec((1, T + 2 * HALO, 16), lambda i: (i, 0, 0)),
            full(rgcn_weight), full(rgcn_root),
            pl.BlockSpec((1, H), lambda i: (0, 0)),
            full(gcn_rel_w), pl.BlockSpec((1, H), lambda i: (0, 0)),
            full(gcn_root_w), full(skip_w),
            pl.BlockSpec((1, H), lambda i: (0, 0)),
            full(cls_w8), full(cls_b8),
        ],
        out_specs=[
            pl.BlockSpec((T, 8), lambda i: (i, 0)),
            pl.BlockSpec(memory_space=pltpu.SMEM),
        ],
        out_shape=[
            jax.ShapeDtypeStruct((npad, 8), jnp.float32),
            jax.ShapeDtypeStruct((1, 1), jnp.float32),
        ],
    )(xpad, xpad, xpad, meta, rgcn_weight, rgcn_root,
      rgcn_bias.reshape(1, H), gcn_rel_w, gcn_rel_b.reshape(1, H),
      gcn_root_w, skip_w, skip_b.reshape(1, H), cls_w8, cls_b8)

    return (out[:N, :6], loss[0, 0])


# speedup vs baseline: 65.2758x; 1.9412x over previous
import functools

import numpy as np
import jax
import jax.numpy as jnp
from jax.experimental import pallas as pl
from jax.experimental.pallas import tpu as pltpu

W = 4
HALO = 2 * W


def _static_meta(B, N, T, nb):
    lengths = np.arange(B, dtype=np.int64)
    lens_node = np.repeat(lengths, lengths)[:N]
    starts = np.cumsum(lengths) - lengths
    starts_node = np.repeat(starts, lengths)[:N]
    pos = np.arange(N, dtype=np.int64) - starts_node

    g = (np.arange(nb)[:, None] * T + np.arange(T + 2 * HALO)[None, :]) - HALO
    ok = (g >= 0) & (g < N)
    gc = np.clip(g, 0, N - 1)
    meta = np.zeros((nb, T + 2 * HALO, 3), dtype=np.float32)
    meta[..., 0] = ok
    meta[..., 1] = pos[gc] * ok
    meta[..., 2] = lens_node[gc] * ok
    return meta, ok


def _windows(v, T, nb, npad):
    vpad = jnp.zeros((npad + 2 * HALO,), v.dtype).at[HALO:HALO + v.shape[0]].set(v)
    main = vpad[HALO:HALO + nb * T].reshape(nb, T)
    left = vpad[:nb * T].reshape(nb, T)[:, :HALO]
    right = jnp.concatenate([vpad[HALO + T:], jnp.zeros((T,), v.dtype)])[
        :nb * T].reshape(nb, T)[:, :HALO]
    return jnp.concatenate([left, main, right], axis=1)


def _band_tiles(rhs, pos, ln, row_off, n_rows, dmat_full):
    dn = (((1,), (0,)), ((), ()))
    pieces = []
    for t in range(0, n_rows, 128):
        rt = min(128, n_rows - t)
        ct = rt + 2 * W
        base = t
        if rt == 128:
            dmat = dmat_full
        else:
            dmat = (jax.lax.broadcasted_iota(jnp.int32, (rt, ct), 1)
                    - jax.lax.broadcasted_iota(jnp.int32, (rt, ct), 0) - W)
        p = pos[t + row_off:t + row_off + rt, :].astype(jnp.int32)
        l = ln[t + row_off:t + row_off + rt, :].astype(jnp.int32)
        q = p + dmat
        a = ((dmat >= -W) & (dmat <= W) & (q >= 0) & (q < l)).astype(jnp.float32)
        pieces.append(jax.lax.dot_general(
            a, rhs[base:base + ct, :], dn, preferred_element_type=jnp.float32))
    return jnp.concatenate(pieces, axis=0) if len(pieces) > 1 else pieces[0]


def _diagcn_block(xm_ref, xc_ref, xp_ref, meta_ref, wcat_ref, rb_ref,
                  relw_ref, relb_ref, rootw_ref, skw_ref, skb_ref,
                  cw_ref, cb_ref, out_ref, loss_ref, *, T, inv_n):
    i = pl.program_id(0)
    e = jnp.concatenate(
        [xm_ref[T - HALO:, :], xc_ref[...], xp_ref[:HALO, :]], axis=0)
    meta = meta_ref[0]
    sp = meta[:, 0:1]
    pos = meta[:, 3:4]
    ln = meta[:, 4:5]
    xs = e * sp

    Th = T + 2 * W
    dmat_full = (jax.lax.broadcasted_iota(jnp.int32, (128, 136), 1)
                 - jax.lax.broadcasted_iota(jnp.int32, (128, 136), 0) - W)

    e2 = jnp.concatenate([e, xs], axis=1)
    w2 = jnp.concatenate([jnp.ones_like(sp), sp], axis=1)
    s = _band_tiles(e2, pos, ln, W, Th, dmat_full)
    c = _band_tiles(w2, pos, ln, W, Th, dmat_full)
    s_all = s[:, :128]
    s_sp = s[:, 128:]
    c_all = c[:, 0:1]
    c_sp = c[:, 1:2]

    sp_h = sp[W:W + Th, :]
    cnt1 = sp_h * c_sp
    r1 = sp_h / jnp.maximum(cnt1, 1.0)
    r0 = 1.0 / jnp.maximum(c_all - cnt1, 1.0)
    mean1 = r1 * s_sp
    mean0 = r0 * s_all - (r0 * sp_h) * s_sp
    xh = e[W:W + Th, :]

    dn_nn = (((1,), (0,)), ((), ()))
    dn_tn = (((0,), (1,)), ((), ()))
    lhs = jnp.concatenate([mean0, mean1, xh], axis=1)
    h = (jax.lax.dot_general(lhs, wcat_ref[...], dn_nn,
                             preferred_element_type=jnp.float32)
         + rb_ref[...])

    m_rel = jax.lax.dot_general(relw_ref[...], cw_ref[...], dn_tn,
                                preferred_element_type=jnp.float32)
    m_root = jax.lax.dot_general(rootw_ref[...], cw_ref[...], dn_tn,
                                 preferred_element_type=jnp.float32)
    m_skip = jax.lax.dot_general(skw_ref[...], cw_ref[...], dn_tn,
                                 preferred_element_type=jnp.float32)
    bias2 = relb_ref[...] + skb_ref[...]
    const_row = (jax.lax.dot_general(bias2, cw_ref[...],
                                     (((1,), (1,)), ((), ())),
                                     preferred_element_type=jnp.float32)
                 + cb_ref[...])

    hm = jax.lax.dot_general(h, m_rel, dn_nn,
                             preferred_element_type=jnp.float32)
    agg8 = _band_tiles(hm, pos, ln, HALO, T, dmat_full)
    hc = h[W:W + T, :]
    xc = e[HALO:HALO + T, :]
    out = (agg8
           + jax.lax.dot_general(hc, m_root, dn_nn,
                                 preferred_element_type=jnp.float32)
           + jax.lax.dot_general(xc, m_skip, dn_nn,
                                 preferred_element_type=jnp.float32)
           + const_row)
    out_ref[...] = out

    lane = jax.lax.broadcasted_iota(jnp.int32, out.shape, 1)
    neg = jnp.float32(-1e30)
    outm = jnp.where(lane < 6, out, neg)
    mx = jnp.max(outm, axis=1, keepdims=True)
    lse = mx + jnp.log(jnp.sum(jnp.exp(outm - mx), axis=1, keepdims=True))
    lab = meta[HALO:HALO + T, 1:2].astype(jnp.int32)
    rv = meta[HALO:HALO + T, 2:3]
    picked = jnp.sum(jnp.where(lane == lab, out, 0.0), axis=1, keepdims=True)
    partial = jnp.sum(rv * (lse - picked)) * inv_n

    @pl.when(i == 0)
    def _():
        loss_ref[0, 0] = partial

    @pl.when(i > 0)
    def _():
        loss_ref[0, 0] = loss_ref[0, 0] + partial


def kernel(input, dialog_lengths, speakers, labels, rgcn_weight, rgcn_root,
           rgcn_bias, gcn_rel_w, gcn_rel_b, gcn_root_w, skip_w, skip_b,
           cls_w, cls_b):
    B = dialog_lengths.shape[0]
    N, D = input.shape
    H = rgcn_root.shape[1]
    T = 2048
    nb = -(-N // T)
    npad = nb * T

    meta_np, ok = _static_meta(B, N, T, nb)
    okf = jnp.asarray(ok.astype(np.float32))
    spw = _windows(speakers.astype(jnp.float32), T, nb, npad) * okf
    labw = _windows(labels.astype(jnp.float32), T, nb, npad) * okf
    meta = jnp.concatenate(
        [spw[..., None], labw[..., None], jnp.asarray(meta_np),
         jnp.zeros((nb, T + 2 * HALO, 3), jnp.float32)], axis=2)

    xpad = jnp.zeros((npad, D), jnp.float32).at[:N].set(input)
    wcat = jnp.concatenate([rgcn_weight[0], rgcn_weight[1], rgcn_root], axis=0)
    cls_w8 = jnp.zeros((8, H), jnp.float32).at[:6].set(cls_w)
    cls_b8 = jnp.zeros((1, 8), jnp.float32).at[0, :6].set(cls_b)

    row_spec = lambda f: pl.BlockSpec((T, D), lambda i: (f(i), 0))
    full = lambda a: pl.BlockSpec(a.shape, lambda i: (0,) * a.ndim)

    out, loss = pl.pallas_call(
        functools.partial(_diagcn_block, T=T, inv_n=1.0 / N),
        grid=(nb,),
        in_specs=[
            row_spec(lambda i: jnp.maximum(i - 1, 0)),
            row_spec(lambda i: i),
            row_spec(lambda i: jnp.minimum(i + 1, nb - 1)),
            pl.BlockSpec((1, T + 2 * HALO, 8), lambda i: (i, 0, 0)),
            full(wcat),
            pl.BlockSpec((1, H), lambda i: (0, 0)),
            full(gcn_rel_w), pl.BlockSpec((1, H), lambda i: (0, 0)),
            full(gcn_root_w), full(skip_w),
            pl.BlockSpec((1, H), lambda i: (0, 0)),
            full(cls_w8), full(cls_b8),
        ],
        out_specs=[
            pl.BlockSpec((T, 8), lambda i: (i, 0)),
            pl.BlockSpec(memory_space=pltpu.SMEM),
        ],
        out_shape=[
            jax.ShapeDtypeStruct((npad, 8), jnp.float32),
            jax.ShapeDtypeStruct((1, 1), jnp.float32),
        ],
    )(xpad, xpad, xpad, meta, wcat, rgcn_bias.reshape(1, H),
      gcn_rel_w, gcn_rel_b.reshape(1, H), gcn_root_w, skip_w,
      skip_b.reshape(1, H), cls_w8, cls_b8)

    return (out[:N, :6], loss[0, 0])
